# baseline (speedup 1.0000x reference)
import functools

import jax
import jax.numpy as jnp
from jax import lax
from jax.experimental import pallas as pl
from jax.experimental.pallas import tpu as pltpu
from jax.experimental.pallas import tpu_sc as plsc

B = 16384
EMB = 128
NC = 2
NS = 16
L = 16
NW = NC * NS
BPW = B // NW
CH = 16
NCH = BPW // CH
NSLOT = 8
NCK = EMB // L


def _build():
    mesh = plsc.VectorSubcoreMesh(core_axis_name="c", subcore_axis_name="s")

    @functools.partial(
        pl.kernel,
        mesh=mesh,
        out_type=(
            jax.ShapeDtypeStruct((B,), jnp.float32),
            jax.ShapeDtypeStruct((B,), jnp.float32),
        ),
        scratch_types=[
            pltpu.VMEM((BPW,), jnp.int32),
            pltpu.VMEM((BPW,), jnp.int32),
            pltpu.VMEM((BPW,), jnp.int32),
            pltpu.VMEM((NSLOT * CH, EMB), jnp.float32),
            pltpu.VMEM((NSLOT * CH, EMB), jnp.float32),
            pltpu.VMEM((NSLOT * CH, EMB), jnp.float32),
            pltpu.VMEM((BPW,), jnp.float32),
            pltpu.VMEM((BPW,), jnp.float32),
            pltpu.SemaphoreType.DMA,
            pltpu.SemaphoreType.DMA((NSLOT,)),
        ],
    )
    def bprmf(user, pos, neg, utab, itab, pos_out, neg_out,
              uidx, pidx, nidx, ub, pb, nb, pov, nov, isem, sems):
        wid = lax.axis_index("s") * NC + lax.axis_index("c")
        base = wid * BPW
        hu = pltpu.async_copy(user.at[pl.ds(base, BPW)], uidx, isem)
        hp = pltpu.async_copy(pos.at[pl.ds(base, BPW)], pidx, isem)
        hn = pltpu.async_copy(neg.at[pl.ds(base, BPW)], nidx, isem)
        hu.wait()
        hp.wait()
        hn.wait()

        def fire(c):
            slot = lax.rem(c, NSLOT)
            s = sems.at[slot]
            off = pl.ds(c * CH, CH)
            dst = pl.ds(slot * CH, CH)
            pltpu.async_copy(utab.at[uidx.at[off]], ub.at[dst], s)
            pltpu.async_copy(itab.at[pidx.at[off]], pb.at[dst], s)
            pltpu.async_copy(itab.at[nidx.at[off]], nb.at[dst], s)

        def drain(c):
            slot = lax.rem(c, NSLOT)
            s = sems.at[slot]
            dst = pl.ds(slot * CH, CH)
            pltpu.make_async_copy(utab.at[uidx.at[pl.ds(0, CH)]], ub.at[dst], s).wait()
            pltpu.make_async_copy(itab.at[pidx.at[pl.ds(0, CH)]], pb.at[dst], s).wait()
            pltpu.make_async_copy(itab.at[nidx.at[pl.ds(0, CH)]], nb.at[dst], s).wait()

        lanes = lax.iota(jnp.int32, L)

        def hsum(acc):
            a8 = lax.slice(acc, (0,), (8,)) + lax.slice(acc, (8,), (16,))
            a4 = lax.slice(a8, (0,), (4,)) + lax.slice(a8, (4,), (8,))
            a2 = lax.slice(a4, (0,), (2,)) + lax.slice(a4, (2,), (4,))
            a1 = lax.slice(a2, (0,), (1,)) + lax.slice(a2, (1,), (2,))
            return lax.reshape(a1, ())

        def compute_chunk(c):
            slot = lax.rem(c, NSLOT)
            b0 = slot * CH

            def row4(q, carry):
                pvec, nvec = carry
                for u in range(4):
                    rr = q * 4 + u
                    r = b0 + rr
                    accp = jnp.zeros((L,), jnp.float32)
                    accn = jnp.zeros((L,), jnp.float32)
                    for ck in range(NCK):
                        sl = pl.ds(ck * L, L)
                        uc = ub[r, sl]
                        accp = accp + uc * pb[r, sl]
                        accn = accn + uc * nb[r, sl]
                    hit = lanes == rr
                    pvec = jnp.where(hit, hsum(accp), pvec)
                    nvec = jnp.where(hit, hsum(accn), nvec)
                return (pvec, nvec)

            pvec, nvec = lax.fori_loop(
                0, L // 4, row4,
                (jnp.zeros((L,), jnp.float32), jnp.zeros((L,), jnp.float32)),
            )
            o = c * CH
            pov[pl.ds(o, L)] = pvec
            nov[pl.ds(o, L)] = nvec

        def prime(c, _):
            fire(c)
            return 0

        lax.fori_loop(0, NSLOT - 1, prime, 0)

        def chunk_body(c, _):
            @pl.when(c + NSLOT - 1 < NCH)
            def _():
                fire(c + NSLOT - 1)

            drain(c)
            compute_chunk(c)
            return 0

        lax.fori_loop(0, NCH, chunk_body, 0)

        pltpu.sync_copy(pov, pos_out.at[pl.ds(base, BPW)])
        pltpu.sync_copy(nov, neg_out.at[pl.ds(base, BPW)])

    return bprmf


_bprmf = _build()


def kernel(user, pos_item, neg_item, user_table, item_table):
    return _bprmf(
        user.astype(jnp.int32),
        pos_item.astype(jnp.int32),
        neg_item.astype(jnp.int32),
        user_table,
        item_table,
    )


# speedup vs baseline: 1.0240x; 1.0240x over previous
import functools

import jax
import jax.numpy as jnp
from jax import lax
from jax.experimental import pallas as pl
from jax.experimental.pallas import tpu as pltpu
from jax.experimental.pallas import tpu_sc as plsc

B = 16384
EMB = 128
NC = 2
NS = 16
L = 16
NW = NC * NS
BPW = B // NW
CH = 16
NCH = BPW // CH
NSLOT = 8
NCK = EMB // L


def _build():
    mesh = plsc.VectorSubcoreMesh(core_axis_name="c", subcore_axis_name="s")

    @functools.partial(
        pl.kernel,
        mesh=mesh,
        out_type=(
            jax.ShapeDtypeStruct((B,), jnp.float32),
            jax.ShapeDtypeStruct((B,), jnp.float32),
        ),
        scratch_types=[
            pltpu.VMEM((BPW,), jnp.int32),
            pltpu.VMEM((BPW,), jnp.int32),
            pltpu.VMEM((BPW,), jnp.int32),
            pltpu.VMEM((NSLOT * CH, EMB), jnp.float32),
            pltpu.VMEM((NSLOT * CH, EMB), jnp.float32),
            pltpu.VMEM((NSLOT * CH, EMB), jnp.float32),
            pltpu.VMEM((BPW,), jnp.float32),
            pltpu.VMEM((BPW,), jnp.float32),
            pltpu.SemaphoreType.DMA,
            pltpu.SemaphoreType.DMA((NSLOT,)),
        ],
    )
    def bprmf(user, pos, neg, utab, itab, pos_out, neg_out,
              uidx, pidx, nidx, ub, pb, nb, pov, nov, isem, sems):
        wid = lax.axis_index("s") * NC + lax.axis_index("c")
        base = wid * BPW
        hu = pltpu.async_copy(user.at[pl.ds(base, BPW)], uidx, isem)
        hp = pltpu.async_copy(pos.at[pl.ds(base, BPW)], pidx, isem)
        hn = pltpu.async_copy(neg.at[pl.ds(base, BPW)], nidx, isem)
        hu.wait()
        hp.wait()
        hn.wait()

        def fire(c):
            slot = lax.rem(c, NSLOT)
            s = sems.at[slot]
            off = pl.ds(c * CH, CH)
            dst = pl.ds(slot * CH, CH)
            pltpu.async_copy(utab.at[uidx.at[off]], ub.at[dst], s)
            pltpu.async_copy(itab.at[pidx.at[off]], pb.at[dst], s)
            pltpu.async_copy(itab.at[nidx.at[off]], nb.at[dst], s)

        def drain(c):
            slot = lax.rem(c, NSLOT)
            s = sems.at[slot]
            dst = pl.ds(slot * CH, CH)
            pltpu.make_async_copy(utab.at[uidx.at[pl.ds(0, CH)]], ub.at[dst], s).wait()
            pltpu.make_async_copy(itab.at[pidx.at[pl.ds(0, CH)]], pb.at[dst], s).wait()
            pltpu.make_async_copy(itab.at[nidx.at[pl.ds(0, CH)]], nb.at[dst], s).wait()

        lanes = lax.iota(jnp.int32, L)

        def hsum(acc):
            a8 = lax.slice(acc, (0,), (8,)) + lax.slice(acc, (8,), (16,))
            a4 = lax.slice(a8, (0,), (4,)) + lax.slice(a8, (4,), (8,))
            a2 = lax.slice(a4, (0,), (2,)) + lax.slice(a4, (2,), (4,))
            a1 = lax.slice(a2, (0,), (1,)) + lax.slice(a2, (1,), (2,))
            return lax.reshape(a1, ())

        def compute_chunk(c):
            slot = lax.rem(c, NSLOT)
            b0 = slot * CH

            def row(rr, carry):
                pvec, nvec = carry
                r = b0 + rr
                accp = jnp.zeros((L,), jnp.float32)
                accn = jnp.zeros((L,), jnp.float32)
                for ck in range(NCK):
                    sl = pl.ds(ck * L, L)
                    uc = ub[r, sl]
                    accp = accp + uc * pb[r, sl]
                    accn = accn + uc * nb[r, sl]
                hit = lanes == rr
                pvec = jnp.where(hit, hsum(accp), pvec)
                nvec = jnp.where(hit, hsum(accn), nvec)
                return (pvec, nvec)

            pvec, nvec = lax.fori_loop(
                0, L, row,
                (jnp.zeros((L,), jnp.float32), jnp.zeros((L,), jnp.float32)),
            )
            o = c * CH
            pov[pl.ds(o, L)] = pvec
            nov[pl.ds(o, L)] = nvec

        def prime(c, _):
            fire(c)
            return 0

        lax.fori_loop(0, NSLOT - 1, prime, 0)

        def chunk_body(c, _):
            @pl.when(c + NSLOT - 1 < NCH)
            def _():
                fire(c + NSLOT - 1)

            drain(c)
            compute_chunk(c)
            return 0

        lax.fori_loop(0, NCH, chunk_body, 0)

        pltpu.sync_copy(pov, pos_out.at[pl.ds(base, BPW)])
        pltpu.sync_copy(nov, neg_out.at[pl.ds(base, BPW)])

    return bprmf


_bprmf = _build()


def kernel(user, pos_item, neg_item, user_table, item_table):
    return _bprmf(
        user.astype(jnp.int32),
        pos_item.astype(jnp.int32),
        neg_item.astype(jnp.int32),
        user_table,
        item_table,
    )
